# baseline (speedup 1.0000x reference)
import functools
import math

import jax
import jax.numpy as jnp
from jax import lax
from jax.experimental import pallas as pl
from jax.experimental.pallas import tpu as pltpu
from jax.experimental.pallas import tpu_sc as plsc

MODEL_DIM = 64
SCALE = math.sqrt(MODEL_DIM)
LANES = 16
NUM_WORKERS = 32
CHUNK = 128


def _emb_kernel_body(n_chunks, idx_hbm, w_hbm, out_hbm, idx_v, rows_v, sem):
    b_per_w = n_chunks * CHUNK
    cid = lax.axis_index("c")
    sid = lax.axis_index("s")
    wid = sid * 2 + cid
    base = wid * b_per_w
    pltpu.sync_copy(idx_hbm.at[pl.ds(base, b_per_w)], idx_v)

    def chunk_body(c, carry):
        off = c * CHUNK
        idx_sl = idx_v.at[pl.ds(off, CHUNK)]
        pltpu.async_copy(w_hbm.at[idx_sl], rows_v, sem).wait()

        def row_body(r, rcarry):
            for cc in range(MODEL_DIM // LANES):
                sl = pl.ds(cc * LANES, LANES)
                rows_v[r, sl] = rows_v[r, sl] * SCALE
            return rcarry

        lax.fori_loop(0, CHUNK, row_body, 0)
        pltpu.sync_copy(rows_v, out_hbm.at[pl.ds(base + off, CHUNK)])
        return carry

    lax.fori_loop(0, n_chunks, chunk_body, 0)


def kernel(input_ids, weight):
    n_rows, n_cols = input_ids.shape
    total = n_rows * n_cols
    b_per_w = total // NUM_WORKERS
    n_chunks = b_per_w // CHUNK
    assert b_per_w * NUM_WORKERS == total and n_chunks * CHUNK == b_per_w

    idx = input_ids.reshape(total).astype(jnp.int32)

    mesh = plsc.VectorSubcoreMesh(core_axis_name="c", subcore_axis_name="s")
    emb = functools.partial(
        pl.kernel,
        mesh=mesh,
        out_type=jax.ShapeDtypeStruct((total, MODEL_DIM), jnp.float32),
        scratch_types=[
            pltpu.VMEM((b_per_w,), jnp.int32),
            pltpu.VMEM((CHUNK, MODEL_DIM), jnp.float32),
            pltpu.SemaphoreType.DMA,
        ],
        compiler_params=pltpu.CompilerParams(use_tc_tiling_on_sc=False),
    )(functools.partial(_emb_kernel_body, n_chunks))

    out = emb(idx, weight)
    return out.reshape(n_rows, n_cols, MODEL_DIM)


# speedup vs baseline: 1.2081x; 1.2081x over previous
import functools
import math

import jax
import jax.numpy as jnp
from jax import lax
from jax.experimental import pallas as pl
from jax.experimental.pallas import tpu as pltpu
from jax.experimental.pallas import tpu_sc as plsc

MODEL_DIM = 64
SCALE = math.sqrt(MODEL_DIM)
LANES = 16
NUM_WORKERS = 32
CHUNK = 128
NG = 4
NS = 2
ROW_UNROLL = 4


def _emb_kernel_body(n_chunks, idx_hbm, w_hbm, out_hbm, idx_v, rows_g, rows_s,
                     gs0, gs1, gs2, gs3, ss0, ss1):
    b_per_w = n_chunks * CHUNK
    gsems = (gs0, gs1, gs2, gs3)
    ssems = (ss0, ss1)
    cid = lax.axis_index("c")
    sid = lax.axis_index("s")
    wid = sid * 2 + cid
    base = wid * b_per_w
    pltpu.sync_copy(idx_hbm.at[pl.ds(base, b_per_w)], idx_v)

    def gather_cp(c, b):
        idx_sl = idx_v.at[pl.ds(c * CHUNK, CHUNK)]
        return pltpu.make_async_copy(w_hbm.at[idx_sl], rows_g.at[b], gsems[b])

    def store_cp(c, sb):
        dst = out_hbm.at[pl.ds(base + c * CHUNK, CHUNK)]
        return pltpu.make_async_copy(rows_s.at[sb], dst, ssems[sb])

    for b in range(NG):
        gather_cp(b, b).start()

    def group_body(g, carry):
        for b in range(NG):
            c = g * NG + b
            sb = b % NS
            if b < NS:
                @pl.when(g > 0)
                def _wait_prev_store():
                    store_cp(c - NS, sb).wait()
            else:
                store_cp(c - NS, sb).wait()
            gather_cp(c, b).wait()
            rg = rows_g.at[b]
            rs = rows_s.at[sb]

            def row_body(r0, rcarry):
                for rr in range(ROW_UNROLL):
                    r = r0 * ROW_UNROLL + rr
                    for cc in range(MODEL_DIM // LANES):
                        sl = pl.ds(cc * LANES, LANES)
                        rs[r, sl] = rg[r, sl] * SCALE
                return rcarry

            lax.fori_loop(0, CHUNK // ROW_UNROLL, row_body, 0)
            store_cp(c, sb).start()
            @pl.when(c + NG < n_chunks)
            def _refill():
                gather_cp(c + NG, b).start()
        return carry

    lax.fori_loop(0, n_chunks // NG, group_body, 0)
    for sb in range(NS):
        store_cp(n_chunks - NS + sb, (n_chunks - NS + sb) % NS).wait()


def kernel(input_ids, weight):
    n_rows, n_cols = input_ids.shape
    total = n_rows * n_cols
    b_per_w = total // NUM_WORKERS
    n_chunks = b_per_w // CHUNK
    assert b_per_w * NUM_WORKERS == total
    assert n_chunks * CHUNK == b_per_w and n_chunks % NG == 0

    idx = input_ids.reshape(total).astype(jnp.int32)

    mesh = plsc.VectorSubcoreMesh(core_axis_name="c", subcore_axis_name="s")
    emb = functools.partial(
        pl.kernel,
        mesh=mesh,
        out_type=jax.ShapeDtypeStruct((total, MODEL_DIM), jnp.float32),
        scratch_types=[
            pltpu.VMEM((b_per_w,), jnp.int32),
            pltpu.VMEM((NG, CHUNK, MODEL_DIM), jnp.float32),
            pltpu.VMEM((NS, CHUNK, MODEL_DIM), jnp.float32),
        ] + [pltpu.SemaphoreType.DMA] * (NG + NS),
        compiler_params=pltpu.CompilerParams(use_tc_tiling_on_sc=False),
    )(functools.partial(_emb_kernel_body, n_chunks))

    out = emb(idx, weight)
    return out.reshape(n_rows, n_cols, MODEL_DIM)
